# Initial kernel scaffold; baseline (speedup 1.0000x reference)
import jax
import jax.numpy as jnp
from jax.experimental import pallas as pl


def kernel(x, gate_w, fc1_w, fc1_b, fc2_w, fc2_b, sfc1_w, sfc1_b, sfc2_w, sfc2_b):
    raise NotImplementedError("write your pallas kernel here")


# speedup vs baseline: 1.8913x; 1.8913x over previous
import functools

import jax
import jax.numpy as jnp
from jax import lax
from jax.experimental import pallas as pl
from jax.experimental.pallas import tpu as pltpu
from jax.experimental.pallas import tpu_sc as plsc

_E = 64
_TOPK = 2
_D = 768
_INTER = 256
_BLK = 256
_NB = 128
_GATE_TB = 512
_TB = 512


def _gate_body(x_ref, gw_ref, idx_ref, w_ref):
    x = x_ref[...]
    g = gw_ref[...]
    logits = lax.dot_general(x, g, (((1,), (1,)), ((), ())),
                             preferred_element_type=jnp.float32)
    lanes = lax.broadcasted_iota(jnp.int32, logits.shape, 1)
    m1 = jnp.max(logits, axis=1, keepdims=True)
    i1 = jnp.min(jnp.where(logits == m1, lanes, _E), axis=1, keepdims=True)
    masked = jnp.where(lanes == i1, -jnp.inf, logits)
    m2 = jnp.max(masked, axis=1, keepdims=True)
    i2 = jnp.min(jnp.where(masked == m2, lanes, _E), axis=1, keepdims=True)
    w1 = 1.0 / (1.0 + jnp.exp(m2 - m1))
    w2 = 1.0 - w1
    idx_ref[...] = jnp.concatenate([i1, i2], axis=1)
    w_ref[...] = jnp.concatenate([w1, w2], axis=1)


def _gate(xf, gate_w):
    n = xf.shape[0]
    grid = (n // _GATE_TB,)
    return pl.pallas_call(
        _gate_body,
        grid=grid,
        in_specs=[
            pl.BlockSpec((_GATE_TB, _D), lambda b: (b, 0)),
            pl.BlockSpec((_E, _D), lambda b: (0, 0)),
        ],
        out_specs=[
            pl.BlockSpec((_GATE_TB, _TOPK), lambda b: (b, 0)),
            pl.BlockSpec((_GATE_TB, _TOPK), lambda b: (b, 0)),
        ],
        out_shape=[
            jax.ShapeDtypeStruct((n, _TOPK), jnp.int32),
            jax.ShapeDtypeStruct((n, _TOPK), jnp.float32),
        ],
    )(xf, gate_w)


def _sc_gather_rows(table, idx):
    b, = idx.shape
    v, d = table.shape
    nw = 32
    ch = 128
    per_w = b // nw
    n_ch = per_w // ch
    mesh = plsc.VectorSubcoreMesh(core_axis_name="c", subcore_axis_name="s")

    @functools.partial(
        pl.kernel,
        mesh=mesh,
        out_type=jax.ShapeDtypeStruct((b, d), table.dtype),
        scratch_types=[
            pltpu.VMEM((ch,), jnp.int32),
            pltpu.VMEM((ch, d), table.dtype),
            pltpu.SemaphoreType.DMA,
        ],
    )
    def k(table_hbm, idx_hbm, out_hbm, idx_v, rows_v, sem):
        wid = lax.axis_index("s") * 2 + lax.axis_index("c")

        def body(c, carry):
            base = wid * per_w + c * ch
            pltpu.sync_copy(idx_hbm.at[pl.ds(base, ch)], idx_v)
            pltpu.async_copy(table_hbm.at[idx_v], rows_v, sem).wait()
            pltpu.sync_copy(rows_v, out_hbm.at[pl.ds(base, ch)])
            return carry

        lax.fori_loop(0, n_ch, body, 0)

    return k(table, idx)


def _ggemm_body(be_ref, x_ref, w1_ref, b1_ref, w2_ref, b2_ref, wp_ref, out_ref):
    x = x_ref[...]
    w = wp_ref[...]
    x = jnp.where(w > 0.0, x, 0.0)
    w1 = w1_ref[0]
    h = lax.dot_general(x, w1, (((1,), (1,)), ((), ())),
                        preferred_element_type=jnp.float32)
    h = h + b1_ref[0]
    h = 0.5 * h * (1.0 + lax.erf(h * 0.7071067811865476))
    w2 = w2_ref[0]
    y = lax.dot_general(h, w2, (((1,), (1,)), ((), ())),
                        preferred_element_type=jnp.float32)
    y = (y + b2_ref[0]) * w
    out_ref[...] = y


def _grouped_gemm(xg, fc1_w, fc1_b, fc2_w, fc2_b, w_pad, block_expert):
    grid_spec = pltpu.PrefetchScalarGridSpec(
        num_scalar_prefetch=1,
        grid=(_NB,),
        in_specs=[
            pl.BlockSpec((_BLK, _D), lambda b, be: (b, 0)),
            pl.BlockSpec((1, _INTER, _D), lambda b, be: (be[b], 0, 0)),
            pl.BlockSpec((1, 1, _INTER), lambda b, be: (be[b], 0, 0)),
            pl.BlockSpec((1, _D, _INTER), lambda b, be: (be[b], 0, 0)),
            pl.BlockSpec((1, 1, _D), lambda b, be: (be[b], 0, 0)),
            pl.BlockSpec((_BLK, 1), lambda b, be: (b, 0)),
        ],
        out_specs=pl.BlockSpec((_BLK, _D), lambda b, be: (b, 0)),
    )
    return pl.pallas_call(
        _ggemm_body,
        grid_spec=grid_spec,
        out_shape=jax.ShapeDtypeStruct((_NB * _BLK, _D), jnp.float32),
    )(block_expert, xg, fc1_w, fc1_b, fc2_w, fc2_b, w_pad)


def _shared_body(x_ref, w1_ref, b1_ref, w2_ref, b2_ref, yp_ref, out_ref):
    x = x_ref[...]
    h = lax.dot_general(x, w1_ref[...], (((1,), (1,)), ((), ())),
                        preferred_element_type=jnp.float32)
    h = h + b1_ref[...]
    h = 0.5 * h * (1.0 + lax.erf(h * 0.7071067811865476))
    s = lax.dot_general(h, w2_ref[...], (((1,), (1,)), ((), ())),
                        preferred_element_type=jnp.float32)
    s = s + b2_ref[...]
    yp = yp_ref[...]
    out_ref[...] = s + yp[:, :_D] + yp[:, _D:]


def _shared_combine(xf, sfc1_w, sfc1_b, sfc2_w, sfc2_b, ypair):
    n = xf.shape[0]
    grid = (n // _TB,)
    return pl.pallas_call(
        _shared_body,
        grid=grid,
        in_specs=[
            pl.BlockSpec((_TB, _D), lambda b: (b, 0)),
            pl.BlockSpec((_INTER, _D), lambda b: (0, 0)),
            pl.BlockSpec((1, _INTER), lambda b: (0, 0)),
            pl.BlockSpec((_D, _INTER), lambda b: (0, 0)),
            pl.BlockSpec((1, _D), lambda b: (0, 0)),
            pl.BlockSpec((_TB, _TOPK * _D), lambda b: (b, 0)),
        ],
        out_specs=pl.BlockSpec((_TB, _D), lambda b: (b, 0)),
        out_shape=jax.ShapeDtypeStruct((n, _D), jnp.float32),
    )(xf, sfc1_w, sfc1_b, sfc2_w, sfc2_b, ypair)


def kernel(x, gate_w, fc1_w, fc1_b, fc2_w, fc2_b, sfc1_w, sfc1_b, sfc2_w, sfc2_b):
    bb, hh, ww, dm = x.shape
    n = bb * hh * ww
    p = n * _TOPK
    xf = x.reshape(n, dm)

    top_idx, top_w = _gate(xf, gate_w)

    e_pair = top_idx.reshape(-1)
    order = jnp.argsort(e_pair, stable=True).astype(jnp.int32)
    e_sorted = e_pair[order]
    tok_sorted = (order // _TOPK).astype(jnp.int32)
    w_sorted = top_w.reshape(-1)[order]
    counts = jnp.bincount(e_pair, length=_E).astype(jnp.int32)
    offs = jnp.cumsum(counts) - counts
    blocks_e = (counts + _BLK - 1) // _BLK
    pad_offs = _BLK * (jnp.cumsum(blocks_e) - blocks_e)
    r = jnp.arange(p, dtype=jnp.int32)
    pp = pad_offs[e_sorted] + (r - offs[e_sorted])
    src_tok = jnp.zeros((_NB * _BLK,), jnp.int32).at[pp].set(tok_sorted)
    w_pad = jnp.zeros((_NB * _BLK, 1), jnp.float32).at[pp, 0].set(w_sorted)
    block_expert = jnp.repeat(jnp.arange(_E, dtype=jnp.int32), blocks_e,
                              total_repeat_length=_NB)
    pair_slot = jnp.zeros((p,), jnp.int32).at[order].set(pp)

    xg = _sc_gather_rows(xf, src_tok)

    yg = _grouped_gemm(xg, fc1_w, fc1_b.reshape(_E, 1, _INTER),
                       fc2_w, fc2_b.reshape(_E, 1, _D), w_pad, block_expert)

    ypair = _sc_gather_rows(yg, pair_slot)

    out = _shared_combine(xf, sfc1_w, sfc1_b.reshape(1, _INTER),
                          sfc2_w, sfc2_b.reshape(1, _D),
                          ypair.reshape(n, _TOPK * _D))
    return out.reshape(bb, hh, ww, dm)


# speedup vs baseline: 3.7720x; 1.9944x over previous
import functools

import jax
import jax.numpy as jnp
from jax import lax
from jax.experimental import pallas as pl
from jax.experimental.pallas import tpu as pltpu
from jax.experimental.pallas import tpu_sc as plsc

_E = 64
_TOPK = 2
_D = 768
_INTER = 256
_BLK = 256
_NB = 128
_GATE_TB = 512
_TB = 512


def _gate_body(x_ref, gw_ref, idx_ref, w_ref):
    x = x_ref[...]
    g = gw_ref[...]
    logits = lax.dot_general(x, g, (((1,), (1,)), ((), ())),
                             preferred_element_type=jnp.float32)
    lanes = lax.broadcasted_iota(jnp.int32, logits.shape, 1)
    m1 = jnp.max(logits, axis=1, keepdims=True)
    i1 = jnp.min(jnp.where(logits == m1, lanes, _E), axis=1, keepdims=True)
    masked = jnp.where(lanes == i1, -jnp.inf, logits)
    m2 = jnp.max(masked, axis=1, keepdims=True)
    i2 = jnp.min(jnp.where(masked == m2, lanes, _E), axis=1, keepdims=True)
    w1 = 1.0 / (1.0 + jnp.exp(m2 - m1))
    w2 = 1.0 - w1
    idx_ref[...] = jnp.concatenate([i1, i2], axis=1)
    w_ref[...] = jnp.concatenate([w1, w2], axis=1)


def _gate(xf, gate_w):
    n = xf.shape[0]
    grid = (n // _GATE_TB,)
    return pl.pallas_call(
        _gate_body,
        grid=grid,
        in_specs=[
            pl.BlockSpec((_GATE_TB, _D), lambda b: (b, 0)),
            pl.BlockSpec((_E, _D), lambda b: (0, 0)),
        ],
        out_specs=[
            pl.BlockSpec((_GATE_TB, _TOPK), lambda b: (b, 0)),
            pl.BlockSpec((_GATE_TB, _TOPK), lambda b: (b, 0)),
        ],
        out_shape=[
            jax.ShapeDtypeStruct((n, _TOPK), jnp.int32),
            jax.ShapeDtypeStruct((n, _TOPK), jnp.float32),
        ],
    )(xf, gate_w)


def _sc_gather_rows(table, idx):
    b, = idx.shape
    v, d = table.shape
    nw = 32
    ch = 128
    per_w = b // nw
    n_ch = per_w // ch
    mesh = plsc.VectorSubcoreMesh(core_axis_name="c", subcore_axis_name="s")

    @functools.partial(
        pl.kernel,
        mesh=mesh,
        out_type=jax.ShapeDtypeStruct((b, d), table.dtype),
        scratch_types=[
            pltpu.VMEM((ch,), jnp.int32),
            pltpu.VMEM((ch, d), table.dtype),
            pltpu.SemaphoreType.DMA,
        ],
    )
    def k(table_hbm, idx_hbm, out_hbm, idx_v, rows_v, sem):
        wid = lax.axis_index("s") * 2 + lax.axis_index("c")

        def body(c, carry):
            base = wid * per_w + c * ch
            pltpu.sync_copy(idx_hbm.at[pl.ds(base, ch)], idx_v)
            pltpu.async_copy(table_hbm.at[idx_v], rows_v, sem).wait()
            pltpu.sync_copy(rows_v, out_hbm.at[pl.ds(base, ch)])
            return carry

        lax.fori_loop(0, n_ch, body, 0)

    return k(table, idx)


def _ggemm_body(be_ref, x_ref, w1_ref, b1_ref, w2_ref, b2_ref, wp_ref, out_ref):
    x = x_ref[...]
    w = wp_ref[...]
    x = jnp.where(w > 0.0, x, 0.0)
    w1 = w1_ref[0]
    h = lax.dot_general(x, w1, (((1,), (1,)), ((), ())),
                        preferred_element_type=jnp.float32)
    h = h + b1_ref[0]
    h = 0.5 * h * (1.0 + lax.erf(h * 0.7071067811865476))
    w2 = w2_ref[0]
    y = lax.dot_general(h, w2, (((1,), (1,)), ((), ())),
                        preferred_element_type=jnp.float32)
    y = (y + b2_ref[0]) * w
    out_ref[...] = y


def _grouped_gemm(xg, fc1_w, fc1_b, fc2_w, fc2_b, w_pad, block_expert):
    grid_spec = pltpu.PrefetchScalarGridSpec(
        num_scalar_prefetch=1,
        grid=(_NB,),
        in_specs=[
            pl.BlockSpec((_BLK, _D), lambda b, be: (b, 0)),
            pl.BlockSpec((1, _INTER, _D), lambda b, be: (be[b], 0, 0)),
            pl.BlockSpec((1, 1, _INTER), lambda b, be: (be[b], 0, 0)),
            pl.BlockSpec((1, _D, _INTER), lambda b, be: (be[b], 0, 0)),
            pl.BlockSpec((1, 1, _D), lambda b, be: (be[b], 0, 0)),
            pl.BlockSpec((_BLK, 1), lambda b, be: (b, 0)),
        ],
        out_specs=pl.BlockSpec((_BLK, _D), lambda b, be: (b, 0)),
    )
    return pl.pallas_call(
        _ggemm_body,
        grid_spec=grid_spec,
        out_shape=jax.ShapeDtypeStruct((_NB * _BLK, _D), jnp.float32),
    )(block_expert, xg, fc1_w, fc1_b, fc2_w, fc2_b, w_pad)


def _shared_body(x_ref, w1_ref, b1_ref, w2_ref, b2_ref, yp_ref, out_ref):
    x = x_ref[...]
    h = lax.dot_general(x, w1_ref[...], (((1,), (1,)), ((), ())),
                        preferred_element_type=jnp.float32)
    h = h + b1_ref[...]
    h = 0.5 * h * (1.0 + lax.erf(h * 0.7071067811865476))
    s = lax.dot_general(h, w2_ref[...], (((1,), (1,)), ((), ())),
                        preferred_element_type=jnp.float32)
    s = s + b2_ref[...]
    yp = yp_ref[...]
    out_ref[...] = s + yp[:, :_D] + yp[:, _D:]


def _shared_combine(xf, sfc1_w, sfc1_b, sfc2_w, sfc2_b, ypair):
    n = xf.shape[0]
    grid = (n // _TB,)
    return pl.pallas_call(
        _shared_body,
        grid=grid,
        in_specs=[
            pl.BlockSpec((_TB, _D), lambda b: (b, 0)),
            pl.BlockSpec((_INTER, _D), lambda b: (0, 0)),
            pl.BlockSpec((1, _INTER), lambda b: (0, 0)),
            pl.BlockSpec((_D, _INTER), lambda b: (0, 0)),
            pl.BlockSpec((1, _D), lambda b: (0, 0)),
            pl.BlockSpec((_TB, _TOPK * _D), lambda b: (b, 0)),
        ],
        out_specs=pl.BlockSpec((_TB, _D), lambda b: (b, 0)),
        out_shape=jax.ShapeDtypeStruct((n, _D), jnp.float32),
    )(xf, sfc1_w, sfc1_b, sfc2_w, sfc2_b, ypair)


def kernel(x, gate_w, fc1_w, fc1_b, fc2_w, fc2_b, sfc1_w, sfc1_b, sfc2_w, sfc2_b):
    bb, hh, ww, dm = x.shape
    n = bb * hh * ww
    p = n * _TOPK
    xf = x.reshape(n, dm)

    top_idx, top_w = _gate(xf, gate_w)

    e_pair = top_idx.reshape(-1)
    order = jnp.argsort(e_pair, stable=True).astype(jnp.int32)
    e_sorted = e_pair[order]
    tok_sorted = (order // _TOPK).astype(jnp.int32)
    w_sorted = top_w.reshape(-1)[order]
    counts = jnp.bincount(e_pair, length=_E).astype(jnp.int32)
    offs = jnp.cumsum(counts) - counts
    blocks_e = (counts + _BLK - 1) // _BLK
    pad_offs = _BLK * (jnp.cumsum(blocks_e) - blocks_e)
    r = jnp.arange(p, dtype=jnp.int32)
    pp = pad_offs[e_sorted] + (r - offs[e_sorted])
    src_tok = (jnp.arange(_NB * _BLK, dtype=jnp.int32) % n).at[pp].set(tok_sorted)
    w_pad = jnp.zeros((_NB * _BLK, 1), jnp.float32).at[pp, 0].set(w_sorted)
    block_expert = jnp.repeat(jnp.arange(_E, dtype=jnp.int32), blocks_e,
                              total_repeat_length=_NB)
    pair_slot = jnp.zeros((p,), jnp.int32).at[order].set(pp)

    xg = _sc_gather_rows(xf, src_tok)

    yg = _grouped_gemm(xg, fc1_w, fc1_b.reshape(_E, 1, _INTER),
                       fc2_w, fc2_b.reshape(_E, 1, _D), w_pad, block_expert)

    ypair = _sc_gather_rows(yg, pair_slot)

    out = _shared_combine(xf, sfc1_w, sfc1_b.reshape(1, _INTER),
                          sfc2_w, sfc2_b.reshape(1, _D),
                          ypair.reshape(n, _TOPK * _D))
    return out.reshape(bb, hh, ww, dm)


# speedup vs baseline: 4.5294x; 1.2008x over previous
import functools

import jax
import jax.numpy as jnp
from jax import lax
from jax.experimental import pallas as pl
from jax.experimental.pallas import tpu as pltpu
from jax.experimental.pallas import tpu_sc as plsc

_E = 64
_TOPK = 2
_D = 768
_INTER = 256
_BLK = 256
_NB = 128
_GATE_TB = 512
_TB = 512


def _gate_body(x_ref, gw_ref, idx_ref, w_ref):
    x = x_ref[...]
    g = gw_ref[...]
    logits = lax.dot_general(x, g, (((1,), (1,)), ((), ())),
                             preferred_element_type=jnp.float32)
    lanes = lax.broadcasted_iota(jnp.int32, logits.shape, 1)
    m1 = jnp.max(logits, axis=1, keepdims=True)
    i1 = jnp.min(jnp.where(logits == m1, lanes, _E), axis=1, keepdims=True)
    masked = jnp.where(lanes == i1, -jnp.inf, logits)
    m2 = jnp.max(masked, axis=1, keepdims=True)
    i2 = jnp.min(jnp.where(masked == m2, lanes, _E), axis=1, keepdims=True)
    w1 = 1.0 / (1.0 + jnp.exp(m2 - m1))
    w2 = 1.0 - w1
    idx_ref[...] = jnp.concatenate([i1, i2], axis=1)
    w_ref[...] = jnp.concatenate([w1, w2], axis=1)


def _gate(xf, gate_w):
    n = xf.shape[0]
    grid = (n // _GATE_TB,)
    return pl.pallas_call(
        _gate_body,
        grid=grid,
        in_specs=[
            pl.BlockSpec((_GATE_TB, _D), lambda b: (b, 0)),
            pl.BlockSpec((_E, _D), lambda b: (0, 0)),
        ],
        out_specs=[
            pl.BlockSpec((_GATE_TB, _TOPK), lambda b: (b, 0)),
            pl.BlockSpec((_GATE_TB, _TOPK), lambda b: (b, 0)),
        ],
        out_shape=[
            jax.ShapeDtypeStruct((n, _TOPK), jnp.int32),
            jax.ShapeDtypeStruct((n, _TOPK), jnp.float32),
        ],
    )(xf, gate_w)


def _sc_gather_rows(table, idx):
    b, = idx.shape
    v, d = table.shape
    nw = 32
    ch = 128
    per_w = b // nw
    n_ch = per_w // ch
    mesh = plsc.VectorSubcoreMesh(core_axis_name="c", subcore_axis_name="s")

    @functools.partial(
        pl.kernel,
        mesh=mesh,
        out_type=jax.ShapeDtypeStruct((b, d), table.dtype),
        scratch_types=[
            pltpu.VMEM((ch,), jnp.int32),
            pltpu.VMEM((ch, d), table.dtype),
            pltpu.SemaphoreType.DMA,
        ],
    )
    def k(table_hbm, idx_hbm, out_hbm, idx_v, rows_v, sem):
        wid = lax.axis_index("s") * 2 + lax.axis_index("c")

        def body(c, carry):
            base = wid * per_w + c * ch
            pltpu.sync_copy(idx_hbm.at[pl.ds(base, ch)], idx_v)
            pltpu.async_copy(table_hbm.at[idx_v], rows_v, sem).wait()
            pltpu.sync_copy(rows_v, out_hbm.at[pl.ds(base, ch)])
            return carry

        lax.fori_loop(0, n_ch, body, 0)

    return k(table, idx)


def _ggemm_body(be_ref, x_ref, w1_ref, b1_ref, w2_ref, b2_ref, wp_ref, out_ref):
    x = x_ref[...]
    w = wp_ref[...]
    x = jnp.where(w > 0.0, x, 0.0)
    w1 = w1_ref[0]
    h = lax.dot_general(x, w1, (((1,), (1,)), ((), ())),
                        preferred_element_type=jnp.float32)
    h = h + b1_ref[0]
    h = 0.5 * h * (1.0 + lax.erf(h * 0.7071067811865476))
    w2 = w2_ref[0]
    y = lax.dot_general(h, w2, (((1,), (1,)), ((), ())),
                        preferred_element_type=jnp.float32)
    y = (y + b2_ref[0]) * w
    out_ref[...] = y


def _grouped_gemm(xg, fc1_w, fc1_b, fc2_w, fc2_b, w_pad, block_expert):
    grid_spec = pltpu.PrefetchScalarGridSpec(
        num_scalar_prefetch=1,
        grid=(_NB,),
        in_specs=[
            pl.BlockSpec((_BLK, _D), lambda b, be: (b, 0)),
            pl.BlockSpec((1, _INTER, _D), lambda b, be: (be[b], 0, 0)),
            pl.BlockSpec((1, 1, _INTER), lambda b, be: (be[b], 0, 0)),
            pl.BlockSpec((1, _D, _INTER), lambda b, be: (be[b], 0, 0)),
            pl.BlockSpec((1, 1, _D), lambda b, be: (be[b], 0, 0)),
            pl.BlockSpec((_BLK, 1), lambda b, be: (b, 0)),
        ],
        out_specs=pl.BlockSpec((_BLK, _D), lambda b, be: (b, 0)),
    )
    return pl.pallas_call(
        _ggemm_body,
        grid_spec=grid_spec,
        out_shape=jax.ShapeDtypeStruct((_NB * _BLK, _D), jnp.float32),
    )(block_expert, xg, fc1_w, fc1_b, fc2_w, fc2_b, w_pad)


def _shared_body(x_ref, w1_ref, b1_ref, w2_ref, b2_ref, yp_ref, out_ref):
    x = x_ref[...]
    h = lax.dot_general(x, w1_ref[...], (((1,), (1,)), ((), ())),
                        preferred_element_type=jnp.float32)
    h = h + b1_ref[...]
    h = 0.5 * h * (1.0 + lax.erf(h * 0.7071067811865476))
    s = lax.dot_general(h, w2_ref[...], (((1,), (1,)), ((), ())),
                        preferred_element_type=jnp.float32)
    s = s + b2_ref[...]
    yp = yp_ref[...]
    out_ref[...] = s + yp[:, :_D] + yp[:, _D:]


def _shared_combine(xf, sfc1_w, sfc1_b, sfc2_w, sfc2_b, ypair):
    n = xf.shape[0]
    grid = (n // _TB,)
    return pl.pallas_call(
        _shared_body,
        grid=grid,
        in_specs=[
            pl.BlockSpec((_TB, _D), lambda b: (b, 0)),
            pl.BlockSpec((_INTER, _D), lambda b: (0, 0)),
            pl.BlockSpec((1, _INTER), lambda b: (0, 0)),
            pl.BlockSpec((_D, _INTER), lambda b: (0, 0)),
            pl.BlockSpec((1, _D), lambda b: (0, 0)),
            pl.BlockSpec((_TB, _TOPK * _D), lambda b: (b, 0)),
        ],
        out_specs=pl.BlockSpec((_TB, _D), lambda b: (b, 0)),
        out_shape=jax.ShapeDtypeStruct((n, _D), jnp.float32),
    )(xf, sfc1_w, sfc1_b, sfc2_w, sfc2_b, ypair)


def kernel(x, gate_w, fc1_w, fc1_b, fc2_w, fc2_b, sfc1_w, sfc1_b, sfc2_w, sfc2_b):
    bb, hh, ww, dm = x.shape
    n = bb * hh * ww
    p = n * _TOPK
    xf = x.reshape(n, dm)

    top_idx, top_w = _gate(xf, gate_w)

    e_pair = top_idx.reshape(-1)
    occ = e_pair[:, None] == jnp.arange(_E, dtype=jnp.int32)
    csum = jnp.cumsum(occ.astype(jnp.int32), axis=0)
    counts = csum[-1]
    rank = jnp.sum(jnp.where(occ, csum, 0), axis=1) - 1
    blocks_e = (counts + _BLK - 1) // _BLK
    pad_offs = _BLK * (jnp.cumsum(blocks_e) - blocks_e)
    pp = pad_offs[e_pair] + rank
    src_tok = (jnp.arange(_NB * _BLK, dtype=jnp.int32) % n).at[pp].set(
        jnp.arange(p, dtype=jnp.int32) // _TOPK)
    w_pad = jnp.zeros((_NB * _BLK, 1), jnp.float32).at[pp, 0].set(top_w.reshape(-1))
    block_expert = jnp.repeat(jnp.arange(_E, dtype=jnp.int32), blocks_e,
                              total_repeat_length=_NB)
    pair_slot = pp

    xg = _sc_gather_rows(xf, src_tok)

    yg = _grouped_gemm(xg, fc1_w, fc1_b.reshape(_E, 1, _INTER),
                       fc2_w, fc2_b.reshape(_E, 1, _D), w_pad, block_expert)

    ypair = _sc_gather_rows(yg, pair_slot)

    out = _shared_combine(xf, sfc1_w, sfc1_b.reshape(1, _INTER),
                          sfc2_w, sfc2_b.reshape(1, _D),
                          ypair.reshape(n, _TOPK * _D))
    return out.reshape(bb, hh, ww, dm)


# speedup vs baseline: 6.3936x; 1.4116x over previous
import functools

import jax
import jax.numpy as jnp
from jax import lax
from jax.experimental import pallas as pl
from jax.experimental.pallas import tpu as pltpu
from jax.experimental.pallas import tpu_sc as plsc

_E = 64
_TOPK = 2
_D = 768
_INTER = 256
_BLK = 256
_NB = 128
_GATE_TB = 512
_TB = 512


def _gate_body(x_ref, gw_ref, idx_ref, w_ref):
    x = x_ref[...]
    g = gw_ref[...]
    logits = lax.dot_general(x, g, (((1,), (1,)), ((), ())),
                             preferred_element_type=jnp.float32)
    lanes = lax.broadcasted_iota(jnp.int32, logits.shape, 1)
    m1 = jnp.max(logits, axis=1, keepdims=True)
    i1 = jnp.min(jnp.where(logits == m1, lanes, _E), axis=1, keepdims=True)
    masked = jnp.where(lanes == i1, -jnp.inf, logits)
    m2 = jnp.max(masked, axis=1, keepdims=True)
    i2 = jnp.min(jnp.where(masked == m2, lanes, _E), axis=1, keepdims=True)
    w1 = 1.0 / (1.0 + jnp.exp(m2 - m1))
    w2 = 1.0 - w1
    idx_ref[...] = jnp.concatenate([i1, i2], axis=1)
    w_ref[...] = jnp.concatenate([w1, w2], axis=1)


def _gate(xf, gate_w):
    n = xf.shape[0]
    grid = (n // _GATE_TB,)
    return pl.pallas_call(
        _gate_body,
        grid=grid,
        in_specs=[
            pl.BlockSpec((_GATE_TB, _D), lambda b: (b, 0)),
            pl.BlockSpec((_E, _D), lambda b: (0, 0)),
        ],
        out_specs=[
            pl.BlockSpec((_GATE_TB, _TOPK), lambda b: (b, 0)),
            pl.BlockSpec((_GATE_TB, _TOPK), lambda b: (b, 0)),
        ],
        out_shape=[
            jax.ShapeDtypeStruct((n, _TOPK), jnp.int32),
            jax.ShapeDtypeStruct((n, _TOPK), jnp.float32),
        ],
    )(xf, gate_w)


def _sc_dispatch(table, tok, pp, out_rows):
    b, = tok.shape
    d = table.shape[1]
    nw = 32
    ch = 128
    per_w = b // nw
    n_ch = per_w // ch
    mesh = plsc.VectorSubcoreMesh(core_axis_name="c", subcore_axis_name="s")

    @functools.partial(
        pl.kernel,
        mesh=mesh,
        out_type=jax.ShapeDtypeStruct((out_rows, d), table.dtype),
        scratch_types=[
            pltpu.VMEM((ch,), jnp.int32),
            pltpu.VMEM((ch,), jnp.int32),
            pltpu.VMEM((ch, d), table.dtype),
            pltpu.SemaphoreType.DMA,
        ],
    )
    def k(table_hbm, tok_hbm, pp_hbm, out_hbm, tok_v, pp_v, rows_v, sem):
        wid = lax.axis_index("s") * 2 + lax.axis_index("c")

        def body(c, carry):
            base = wid * per_w + c * ch
            pltpu.sync_copy(tok_hbm.at[pl.ds(base, ch)], tok_v)
            pltpu.sync_copy(pp_hbm.at[pl.ds(base, ch)], pp_v)
            pltpu.async_copy(table_hbm.at[tok_v], rows_v, sem).wait()
            pltpu.async_copy(rows_v, out_hbm.at[pp_v], sem).wait()
            return carry

        lax.fori_loop(0, n_ch, body, 0)

    return k(table, tok, pp)


def _sc_gather_rows(table, idx):
    b, = idx.shape
    d = table.shape[1]
    nw = 32
    ch = 128
    per_w = b // nw
    n_ch = per_w // ch
    mesh = plsc.VectorSubcoreMesh(core_axis_name="c", subcore_axis_name="s")

    @functools.partial(
        pl.kernel,
        mesh=mesh,
        out_type=jax.ShapeDtypeStruct((b, d), table.dtype),
        scratch_types=[
            pltpu.VMEM((ch,), jnp.int32),
            pltpu.VMEM((ch, d), table.dtype),
            pltpu.SemaphoreType.DMA,
        ],
    )
    def k(table_hbm, idx_hbm, out_hbm, idx_v, rows_v, sem):
        wid = lax.axis_index("s") * 2 + lax.axis_index("c")

        def body(c, carry):
            base = wid * per_w + c * ch
            pltpu.sync_copy(idx_hbm.at[pl.ds(base, ch)], idx_v)
            pltpu.async_copy(table_hbm.at[idx_v], rows_v, sem).wait()
            pltpu.sync_copy(rows_v, out_hbm.at[pl.ds(base, ch)])
            return carry

        lax.fori_loop(0, n_ch, body, 0)

    return k(table, idx)


def _ggemm_body(meta_ref, x_ref, w1_ref, b1_ref, w2_ref, b2_ref, out_ref):
    b = pl.program_id(0)
    blen = meta_ref[_NB + b]
    x = x_ref[...]
    rows = lax.broadcasted_iota(jnp.int32, (_BLK, 1), 0)
    x = jnp.where(rows < blen, x, 0.0)
    xb = x.astype(jnp.bfloat16)
    w1 = w1_ref[0].astype(jnp.bfloat16)
    h = lax.dot_general(xb, w1, (((1,), (1,)), ((), ())),
                        preferred_element_type=jnp.float32)
    h = h + b1_ref[0]
    h = 0.5 * h * (1.0 + lax.erf(h * 0.7071067811865476))
    hb = h.astype(jnp.bfloat16)
    w2 = w2_ref[0].astype(jnp.bfloat16)
    y = lax.dot_general(hb, w2, (((1,), (1,)), ((), ())),
                        preferred_element_type=jnp.float32)
    out_ref[...] = y + b2_ref[0]


def _grouped_gemm(xg, fc1_w, fc1_b, fc2_w, fc2_b, meta):
    grid_spec = pltpu.PrefetchScalarGridSpec(
        num_scalar_prefetch=1,
        grid=(_NB,),
        in_specs=[
            pl.BlockSpec((_BLK, _D), lambda b, m: (b, 0)),
            pl.BlockSpec((1, _INTER, _D), lambda b, m: (m[b], 0, 0)),
            pl.BlockSpec((1, 1, _INTER), lambda b, m: (m[b], 0, 0)),
            pl.BlockSpec((1, _D, _INTER), lambda b, m: (m[b], 0, 0)),
            pl.BlockSpec((1, 1, _D), lambda b, m: (m[b], 0, 0)),
        ],
        out_specs=pl.BlockSpec((_BLK, _D), lambda b, m: (b, 0)),
    )
    return pl.pallas_call(
        _ggemm_body,
        grid_spec=grid_spec,
        out_shape=jax.ShapeDtypeStruct((_NB * _BLK, _D), jnp.float32),
    )(meta, xg, fc1_w, fc1_b, fc2_w, fc2_b)


def _shared_body(x_ref, w1_ref, b1_ref, w2_ref, b2_ref, y0_ref, y1_ref, tw_ref,
                 out_ref):
    x = x_ref[...]
    h = lax.dot_general(x, w1_ref[...], (((1,), (1,)), ((), ())),
                        preferred_element_type=jnp.float32)
    h = h + b1_ref[...]
    h = 0.5 * h * (1.0 + lax.erf(h * 0.7071067811865476))
    s = lax.dot_general(h, w2_ref[...], (((1,), (1,)), ((), ())),
                        preferred_element_type=jnp.float32)
    s = s + b2_ref[...]
    tw = tw_ref[...]
    out_ref[...] = s + tw[:, 0:1] * y0_ref[...] + tw[:, 1:2] * y1_ref[...]


def _shared_combine(xf, sfc1_w, sfc1_b, sfc2_w, sfc2_b, ypair, top_w):
    n = xf.shape[0]
    nblk = n // _TB
    grid = (nblk,)
    return pl.pallas_call(
        _shared_body,
        grid=grid,
        in_specs=[
            pl.BlockSpec((_TB, _D), lambda b: (b, 0)),
            pl.BlockSpec((_INTER, _D), lambda b: (0, 0)),
            pl.BlockSpec((1, _INTER), lambda b: (0, 0)),
            pl.BlockSpec((_D, _INTER), lambda b: (0, 0)),
            pl.BlockSpec((1, _D), lambda b: (0, 0)),
            pl.BlockSpec((_TB, _D), lambda b: (b, 0)),
            pl.BlockSpec((_TB, _D), lambda b, _nb=nblk: (b + _nb, 0)),
            pl.BlockSpec((_TB, _TOPK), lambda b: (b, 0)),
        ],
        out_specs=pl.BlockSpec((_TB, _D), lambda b: (b, 0)),
        out_shape=jax.ShapeDtypeStruct((n, _D), jnp.float32),
    )(xf, sfc1_w, sfc1_b, sfc2_w, sfc2_b, ypair, ypair, top_w)


def kernel(x, gate_w, fc1_w, fc1_b, fc2_w, fc2_b, sfc1_w, sfc1_b, sfc2_w, sfc2_b):
    bb, hh, ww, dm = x.shape
    n = bb * hh * ww
    p = n * _TOPK
    xf = x.reshape(n, dm)

    top_idx, top_w = _gate(xf, gate_w)

    e_pair = top_idx.T.reshape(-1)
    occ = e_pair[:, None] == jnp.arange(_E, dtype=jnp.int32)
    csum = jnp.cumsum(occ.astype(jnp.int32), axis=0)
    counts = csum[-1]
    rank = jnp.sum(jnp.where(occ, csum, 0), axis=1) - 1
    blocks_e = (counts + _BLK - 1) // _BLK
    first_blk = jnp.cumsum(blocks_e) - blocks_e
    pad_offs = _BLK * first_blk
    pp = pad_offs[e_pair] + rank
    block_expert = jnp.repeat(jnp.arange(_E, dtype=jnp.int32), blocks_e,
                              total_repeat_length=_NB)
    block_ord = jnp.arange(_NB, dtype=jnp.int32) - first_blk[block_expert]
    block_len = jnp.clip(counts[block_expert] - block_ord * _BLK, 0, _BLK)
    meta = jnp.concatenate([block_expert, block_len]).astype(jnp.int32)
    tok = jnp.arange(p, dtype=jnp.int32) % n

    xg = _sc_dispatch(xf, tok, pp, _NB * _BLK)

    yg = _grouped_gemm(xg, fc1_w, fc1_b.reshape(_E, 1, _INTER),
                       fc2_w, fc2_b.reshape(_E, 1, _D), meta)

    ypair = _sc_gather_rows(yg, pp)

    out = _shared_combine(xf, sfc1_w, sfc1_b.reshape(1, _INTER),
                          sfc2_w, sfc2_b.reshape(1, _D), ypair, top_w)
    return out.reshape(bb, hh, ww, dm)
